# baseline (speedup 1.0000x reference)
import functools
import jax
import jax.numpy as jnp
from jax import lax
from jax.experimental import pallas as pl
from jax.experimental.pallas import tpu as pltpu, tpu_sc as plsc

_ROWS = 8192
_DIM = 1024
_NC = 2
_NS = 16
_NW = _NC * _NS

_SC_ROWS = 4096
_RPW = _SC_ROWS // _NW
_CHUNK = 32
_NCHUNK = _RPW // _CHUNK
_NBUF = 3 if _NCHUNK >= 3 else 2

_TC_R = 512
_TC_ROWS = _ROWS - _SC_ROWS


def _sc_body(table, idx, out, idx_v, *rest):
    bufs = rest[:_NBUF]
    gsems = rest[_NBUF:2 * _NBUF]
    wsems = rest[2 * _NBUF:3 * _NBUF]
    wid = lax.axis_index("s") * _NC + lax.axis_index("c")
    base = wid * _RPW
    pltpu.sync_copy(idx.at[wid], idx_v)
    g = [None] * _NCHUNK
    w = [None] * _NCHUNK
    for c in range(min(_NBUF, _NCHUNK)):
        g[c] = pltpu.async_copy(table.at[idx_v.at[c]], bufs[c], gsems[c])
    for c in range(_NCHUNK):
        b = c % _NBUF
        g[c].wait()
        w[c] = pltpu.async_copy(
            bufs[b], out.at[pl.ds(base + c * _CHUNK, _CHUNK)], wsems[b])
        n = c + _NBUF
        if n < _NCHUNK:
            w[c].wait()
            g[n] = pltpu.async_copy(table.at[idx_v.at[n]], bufs[b], gsems[b])
    for c in range(max(0, _NCHUNK - _NBUF), _NCHUNK):
        w[c].wait()


def _sc_part(table, idx):
    return pl.kernel(
        _sc_body,
        mesh=plsc.VectorSubcoreMesh(core_axis_name="c", subcore_axis_name="s"),
        out_type=jax.ShapeDtypeStruct((_SC_ROWS, _DIM), jnp.float32),
        scratch_types=(
            [pltpu.VMEM((_NCHUNK, _CHUNK), jnp.int32)]
            + [pltpu.VMEM((_CHUNK, _DIM), jnp.float32)] * _NBUF
            + [pltpu.SemaphoreType.DMA] * (2 * _NBUF)
        ),
    )(table, idx)


_TC_W = _TC_R + 8


def _tc_body(off_ref, table_any, out_ref, win, w2, sems, *, row0, nrows):
    b = pl.program_id(0)
    nb = nrows // _TC_R
    off = off_ref[0]

    def gm_of(blk):
        g = row0 + blk * _TC_R + off
        return lax.rem(lax.rem(g, _ROWS) + _ROWS, _ROWS)

    def win_start(gm):
        return (jnp.clip(gm, 0, _ROWS - _TC_W) // 8) * 8

    def start(blk, slot):
        s1 = win_start(gm_of(blk))
        pltpu.make_async_copy(
            table_any.at[pl.ds(s1, _TC_W)], win.at[slot], sems.at[slot]).start()

    @pl.when(b == 0)
    def _():
        pltpu.make_async_copy(
            table_any.at[pl.ds(0, _TC_R)], w2, sems.at[2]).start()
        start(0, 0)
        pltpu.make_async_copy(
            table_any.at[pl.ds(0, _TC_R)], w2, sems.at[2]).wait()

    @pl.when(b + 1 < nb)
    def _():
        start(b + 1, (b + 1) % 2)

    slot = b % 2
    gm = gm_of(b)
    s1 = win_start(gm)
    d1 = gm - s1
    pltpu.make_async_copy(
        table_any.at[pl.ds(s1, _TC_W)], win.at[slot], sems.at[slot]).wait()

    @pl.when(d1 == 0)
    def _():
        out_ref[...] = win[slot, 0:_TC_R]

    @pl.when(d1 != 0)
    def _():
        W = win[slot]
        rolled1 = pltpu.roll(W, -d1, 0)[0:_TC_R]
        rolled2 = pltpu.roll(w2[...], _ROWS - gm, 0)
        iot = lax.broadcasted_iota(jnp.int32, (_TC_R, 1), 0)
        m1 = jnp.broadcast_to(iot < _ROWS - gm, (_TC_R, _DIM))
        out_ref[...] = jnp.where(m1, rolled1, rolled2)


def _tc_part(offset, table, row0, nrows):
    grid_spec = pltpu.PrefetchScalarGridSpec(
        num_scalar_prefetch=1,
        grid=(nrows // _TC_R,),
        in_specs=[pl.BlockSpec(memory_space=pl.ANY)],
        out_specs=pl.BlockSpec((_TC_R, _DIM), lambda b, off: (b, 0)),
        scratch_shapes=[
            pltpu.VMEM((2, _TC_W, _DIM), jnp.float32),
            pltpu.VMEM((_TC_R, _DIM), jnp.float32),
            pltpu.SemaphoreType.DMA((3,)),
        ],
    )
    return pl.pallas_call(
        functools.partial(_tc_body, row0=row0, nrows=nrows),
        grid_spec=grid_spec,
        out_shape=jax.ShapeDtypeStruct((nrows, _DIM), jnp.float32),
    )(jnp.asarray([offset], jnp.int32), table)


def kernel(seq_len, pos_emb):
    offset = jnp.asarray(seq_len, jnp.int32) - _ROWS
    pos = jnp.arange(_SC_ROWS, dtype=jnp.int32) + offset
    idx = jnp.mod(jnp.mod(pos, _ROWS) + _ROWS, _ROWS).reshape(
        _NW, _NCHUNK, _CHUNK)
    out_sc = _sc_part(pos_emb, idx)
    out_tc = _tc_part(offset, pos_emb, _SC_ROWS, _TC_ROWS)
    return jnp.concatenate([out_sc, out_tc], axis=0)


# speedup vs baseline: 1.2968x; 1.2968x over previous
import jax
import jax.numpy as jnp
from jax import lax
from jax.experimental import pallas as pl
from jax.experimental.pallas import tpu as pltpu, tpu_sc as plsc

_ROWS = 8192
_DIM = 1024
_NC = 2
_NS = 16
_NW = _NC * _NS
_CHUNK = 32
_SPLIT = 2048
_TC_R = 512


def _sc_body(table, idx, out, idx_v, *rest, nchunk, nbuf, rpw):
    bufs = rest[:nbuf]
    gsems = rest[nbuf:2 * nbuf]
    wsems = rest[2 * nbuf:3 * nbuf]
    wid = lax.axis_index("s") * _NC + lax.axis_index("c")
    base = wid * rpw
    pltpu.sync_copy(idx.at[wid], idx_v)
    g = [None] * nchunk
    w = [None] * nchunk
    for c in range(min(nbuf, nchunk)):
        g[c] = pltpu.async_copy(table.at[idx_v.at[c]], bufs[c], gsems[c])
    for c in range(nchunk):
        b = c % nbuf
        g[c].wait()
        w[c] = pltpu.async_copy(
            bufs[b], out.at[pl.ds(base + c * _CHUNK, _CHUNK)], wsems[b])
        n = c + nbuf
        if n < nchunk:
            w[c].wait()
            g[n] = pltpu.async_copy(table.at[idx_v.at[n]], bufs[b], gsems[b])
    for c in range(max(0, nchunk - nbuf), nchunk):
        w[c].wait()


def _sc_gather(table, idx, nrows):
    rpw = nrows // _NW
    nchunk = rpw // _CHUNK
    nbuf = 3 if nchunk >= 3 else max(1, nchunk)

    def body(table_ref, idx_ref, out_ref, idx_v, *rest):
        _sc_body(table_ref, idx_ref, out_ref, idx_v, *rest,
                 nchunk=nchunk, nbuf=nbuf, rpw=rpw)

    return pl.kernel(
        body,
        mesh=plsc.VectorSubcoreMesh(core_axis_name="c", subcore_axis_name="s"),
        out_type=jax.ShapeDtypeStruct((nrows, _DIM), jnp.float32),
        scratch_types=(
            [pltpu.VMEM((nchunk, _CHUNK), jnp.int32)]
            + [pltpu.VMEM((_CHUNK, _DIM), jnp.float32)] * nbuf
            + [pltpu.SemaphoreType.DMA] * (2 * nbuf)
        ),
    )(table, idx)


def _copy_body(x_ref, o_ref):
    o_ref[...] = x_ref[...]


def _tc_tail_copy(table):
    nb = (_ROWS - _SPLIT) // _TC_R
    s0 = _SPLIT // _TC_R
    return pl.pallas_call(
        _copy_body,
        grid=(nb,),
        in_specs=[pl.BlockSpec((_TC_R, _DIM), lambda b: (b + s0, 0))],
        out_specs=pl.BlockSpec((_TC_R, _DIM), lambda b: (b + s0, 0)),
        out_shape=jax.ShapeDtypeStruct((_ROWS, _DIM), jnp.float32),
    )(table)


def _merge_body(a_ref, p_ref, o_ref):
    del p_ref
    o_ref[...] = a_ref[...]


def _tc_merge(sc_out, prev):
    nb = _SPLIT // _TC_R
    return pl.pallas_call(
        _merge_body,
        grid=(nb,),
        in_specs=[
            pl.BlockSpec((_TC_R, _DIM), lambda b: (b, 0)),
            pl.BlockSpec(memory_space=pl.ANY),
        ],
        out_specs=pl.BlockSpec((_TC_R, _DIM), lambda b: (b, 0)),
        out_shape=jax.ShapeDtypeStruct((_ROWS, _DIM), jnp.float32),
        input_output_aliases={1: 0},
    )(sc_out, prev)


def kernel(seq_len, pos_emb):
    offset = jnp.asarray(seq_len, jnp.int32) - _ROWS
    pos = jnp.arange(_ROWS, dtype=jnp.int32) + offset
    idx = jnp.mod(jnp.mod(pos, _ROWS) + _ROWS, _ROWS)
    idx_full = idx.reshape(_NW, _ROWS // _NW // _CHUNK, _CHUNK)
    idx_split = idx[:_SPLIT].reshape(_NW, _SPLIT // _NW // _CHUNK, _CHUNK)

    def fast(operands):
        table, idx_s, _ = operands
        sc_out = _sc_gather(table, idx_s, _SPLIT)
        prev = _tc_tail_copy(table)
        return _tc_merge(sc_out, prev)

    def general(operands):
        table, _, idx_f = operands
        return _sc_gather(table, idx_f, _ROWS)

    return lax.cond(offset == 0, fast, general, (pos_emb, idx_split, idx_full))


# speedup vs baseline: 1.3754x; 1.0606x over previous
import jax
import jax.numpy as jnp
from jax import lax
from jax.experimental import pallas as pl
from jax.experimental.pallas import tpu as pltpu, tpu_sc as plsc

_ROWS = 8192
_DIM = 1024
_NC = 2
_NS = 16
_NW = _NC * _NS
_RPW = _ROWS // _NW
_CHUNK = 32
_NCHUNK = _RPW // _CHUNK
_NBUF = 2


def _sc_body(table, idx, out, idx_v, buf0, buf1, gs0, gs1, ws0, ws1):
    bufs = (buf0, buf1)
    gsems = (gs0, gs1)
    wsems = (ws0, ws1)
    wid = lax.axis_index("s") * _NC + lax.axis_index("c")
    base = wid * _RPW
    pltpu.sync_copy(idx.at[wid], idx_v)

    @pl.loop(0, _NCHUNK, step=_NBUF)
    def _(i):
        gs = []
        for b in range(_NBUF):
            @pl.when(i > 0)
            def _():
                pltpu.make_async_copy(
                    table.at[pl.ds(0, _CHUNK)], bufs[b], wsems[b]).wait()
            gs.append(pltpu.async_copy(
                table.at[idx_v.at[i + b]], bufs[b], gsems[b]))
        for b in range(_NBUF):
            gs[b].wait()
            pltpu.async_copy(
                bufs[b], out.at[pl.ds(base + (i + b) * _CHUNK, _CHUNK)],
                wsems[b])

    for b in range(_NBUF):
        pltpu.make_async_copy(
            table.at[pl.ds(0, _CHUNK)], bufs[b], wsems[b]).wait()


def kernel(seq_len, pos_emb):
    offset = jnp.asarray(seq_len, jnp.int32) - _ROWS
    pos = jnp.arange(_ROWS, dtype=jnp.int32) + offset
    idx = jnp.mod(jnp.mod(pos, _ROWS) + _ROWS, _ROWS).reshape(
        _NW, _NCHUNK, _CHUNK)
    return pl.kernel(
        _sc_body,
        mesh=plsc.VectorSubcoreMesh(core_axis_name="c", subcore_axis_name="s"),
        out_type=jax.ShapeDtypeStruct((_ROWS, _DIM), jnp.float32),
        scratch_types=(
            [pltpu.VMEM((_NCHUNK, _CHUNK), jnp.int32)]
            + [pltpu.VMEM((_CHUNK, _DIM), jnp.float32)] * _NBUF
            + [pltpu.SemaphoreType.DMA] * (2 * _NBUF)
        ),
    )(pos_emb, idx)


# speedup vs baseline: 1.4693x; 1.0683x over previous
import jax
import jax.numpy as jnp
from jax import lax
from jax.experimental import pallas as pl
from jax.experimental.pallas import tpu as pltpu, tpu_sc as plsc

_ROWS = 8192
_DIM = 1024
_NC = 2
_NS = 16
_NW = _NC * _NS
_RPW = _ROWS // _NW
_CHUNK = 16
_NCHUNK = _RPW // _CHUNK
_NBUF = 6


def _sc_body(table, idx, out, idx_v, *rest):
    bufs = rest[:_NBUF]
    gsems = rest[_NBUF:2 * _NBUF]
    wsems = rest[2 * _NBUF:3 * _NBUF]
    wid = lax.axis_index("s") * _NC + lax.axis_index("c")
    base = wid * _RPW
    pltpu.sync_copy(idx.at[wid], idx_v)
    g = [None] * _NCHUNK
    w = [None] * _NCHUNK
    for c in range(min(_NBUF, _NCHUNK)):
        g[c] = pltpu.async_copy(table.at[idx_v.at[c]], bufs[c], gsems[c])
    for c in range(_NCHUNK):
        b = c % _NBUF
        g[c].wait()
        w[c] = pltpu.async_copy(
            bufs[b], out.at[pl.ds(base + c * _CHUNK, _CHUNK)], wsems[b])
        n = c + _NBUF
        if n < _NCHUNK:
            w[c].wait()
            g[n] = pltpu.async_copy(table.at[idx_v.at[n]], bufs[b], gsems[b])
    for c in range(max(0, _NCHUNK - _NBUF), _NCHUNK):
        w[c].wait()


def kernel(seq_len, pos_emb):
    offset = jnp.asarray(seq_len, jnp.int32) - _ROWS
    pos = jnp.arange(_ROWS, dtype=jnp.int32) + offset
    idx = jnp.mod(jnp.mod(pos, _ROWS) + _ROWS, _ROWS).reshape(
        _NW, _NCHUNK, _CHUNK)
    return pl.kernel(
        _sc_body,
        mesh=plsc.VectorSubcoreMesh(core_axis_name="c", subcore_axis_name="s"),
        out_type=jax.ShapeDtypeStruct((_ROWS, _DIM), jnp.float32),
        scratch_types=(
            [pltpu.VMEM((_NCHUNK, _CHUNK), jnp.int32)]
            + [pltpu.VMEM((_CHUNK, _DIM), jnp.float32)] * _NBUF
            + [pltpu.SemaphoreType.DMA] * (2 * _NBUF)
        ),
    )(pos_emb, idx)
